# baseline (speedup 1.0000x reference)
import functools

import jax
import jax.numpy as jnp
from jax import lax
from jax.experimental import pallas as pl
from jax.experimental.pallas import tpu as pltpu
from jax.experimental.pallas import tpu_sc as plsc

NUM_CORES = 2
NUM_SUBCORES = 16
NW = NUM_CORES * NUM_SUBCORES
LANES = 16
NBUF = 8


def _make_sc_linearize(num_fields: int, emb_dim: int, vocab: int):
    full_tiles = vocab // 128
    tail = vocab - full_tiles * 128
    n_items = num_fields * full_tiles
    mesh = plsc.VectorSubcoreMesh(core_axis_name="c", subcore_axis_name="s")

    @functools.partial(
        pl.kernel,
        out_type=jax.ShapeDtypeStruct((num_fields * vocab * emb_dim,),
                                      jnp.float32),
        mesh=mesh,
        scratch_types=(
            [pltpu.VMEM((LANES, 128), jnp.float32) for _ in range(NBUF)]
            + [pltpu.VMEM((128 * LANES,), jnp.float32) for _ in range(NBUF)]
            + [pltpu.VMEM((LANES, 32), jnp.float32)]
            + [pltpu.SemaphoreType.DMA((NBUF,)),
               pltpu.SemaphoreType.DMA((NBUF,))]
        ),
        compiler_params=pltpu.CompilerParams(use_tc_tiling_on_sc=True,
                                             needs_layout_passes=False),
    )
    def sc_lin(tabt_hbm, out_hbm, *scr):
        sin = scr[:NBUF]
        sout = scr[NBUF:2 * NBUF]
        sin_t = scr[2 * NBUF]
        sem_i, sem_o = scr[2 * NBUF + 1], scr[2 * NBUF + 2]
        wid = lax.axis_index("s") * NUM_CORES + lax.axis_index("c")
        n_k = (n_items - wid + NW - 1) // NW
        lane = lax.iota(jnp.int32, LANES)
        lane16 = lane * emb_dim

        def item_fc(k):
            g = wid + k * NW
            return g // full_tiles, g % full_tiles

        def start_in(k, b):
            f, c = item_fc(k)
            pltpu.async_copy(tabt_hbm.at[f, :, pl.ds(c * 128, 128)],
                             sin[b], sem_i.at[b])

        def wait_in(b):
            pltpu.make_async_copy(tabt_hbm.at[0, :, pl.ds(0, 128)],
                                  sin[b], sem_i.at[b]).wait()

        def start_out(k, b):
            f, c = item_fc(k)
            off = (f * vocab + c * 128) * emb_dim
            pltpu.async_copy(sout[b], out_hbm.at[pl.ds(off, 128 * emb_dim)],
                             sem_o.at[b])

        def wait_out(b):
            pltpu.make_async_copy(sout[b],
                                  out_hbm.at[pl.ds(0, 128 * emb_dim)],
                                  sem_o.at[b]).wait()

        for b in range(NBUF):
            @pl.when(b < n_k)
            def _(b=b):
                start_in(b, b)

        def phase(k, b):
            @pl.when(k < n_k)
            def _():
                wait_in(b)

                @pl.when(k >= NBUF)
                def _():
                    wait_out(b)

                def t_body(t, carry):
                    t256 = t * (LANES * emb_dim)
                    for e in range(emb_dim):
                        vals = sin[b][e, pl.ds(t * LANES, LANES)]
                        plsc.store_scatter(sout[b], [lane16 + (t256 + e)],
                                           vals)
                    return carry

                lax.fori_loop(0, 128 // LANES, t_body, 0, unroll=8)
                start_out(k, b)

                @pl.when(k + NBUF < n_k)
                def _():
                    start_in(k + NBUF, b)

        def outer(k0, carry):
            for b in range(NBUF):
                phase(k0 * NBUF + b, b)
            return carry

        max_k = (n_items + NW - 1) // NW
        lax.fori_loop(0, (max_k + NBUF - 1) // NBUF, outer, 0)

        for b in range(NBUF):
            @pl.when(jnp.minimum(n_k, NBUF) > b)
            def _(b=b):
                wait_out(b)

        @pl.when(wid < num_fields)
        def _():
            f = wid
            pltpu.sync_copy(tabt_hbm.at[f, :, pl.ds(full_tiles * 128, tail)],
                            sin_t)

            def tt_body(t, carry):
                t256 = t * (LANES * emb_dim)
                for e in range(emb_dim):
                    vals = sin_t[e, pl.ds(t * LANES, LANES)]
                    plsc.store_scatter(sout[0], [lane16 + (t256 + e)], vals)
                return carry

            lax.fori_loop(0, tail // LANES, tt_body, 0)
            off = (f * vocab + full_tiles * 128) * emb_dim
            pltpu.sync_copy(sout[0].at[pl.ds(0, tail * emb_dim)],
                            out_hbm.at[pl.ds(off, tail * emb_dim)])

    return sc_lin


def _make_sc_gather(b_rows: int, num_fields: int, emb_dim: int, vocab: int):
    chunk = b_rows // NW
    mesh = plsc.VectorSubcoreMesh(core_axis_name="c", subcore_axis_name="s")

    @functools.partial(
        pl.kernel,
        out_type=jax.ShapeDtypeStruct((b_rows, num_fields * emb_dim),
                                      jnp.float32),
        mesh=mesh,
        scratch_types=[
            pltpu.VMEM((chunk, num_fields), jnp.float32),
            pltpu.VMEM((chunk,), jnp.int32),
            pltpu.VMEM((chunk, emb_dim), jnp.float32),
            pltpu.SemaphoreType.DMA,
        ],
        compiler_params=pltpu.CompilerParams(use_tc_tiling_on_sc=False,
                                             needs_layout_passes=False),
    )
    def sc_gather(tab_rows, xcat_hbm, out_hbm, xslab, idx_v, rows_v, sem):
        wid = lax.axis_index("s") * NUM_CORES + lax.axis_index("c")
        base = wid * chunk
        pltpu.sync_copy(xcat_hbm.at[pl.ds(base, chunk)], xslab)
        lane = lax.iota(jnp.int32, LANES)

        def field_body(i, carry):
            col = jnp.full((LANES,), i, jnp.int32)
            off = jnp.zeros((LANES,), jnp.int32) + i * vocab

            def extract_body(t, carry2):
                r16 = lane + t * LANES
                v = plsc.bitcast(plsc.load_gather(xslab, [r16, col]),
                                 jnp.int32)
                idx_v[pl.ds(t * LANES, LANES)] = v + off
                return carry2

            lax.fori_loop(0, chunk // LANES, extract_body, 0)
            pltpu.async_copy(tab_rows.at[idx_v], rows_v, sem).wait()
            pltpu.sync_copy(
                rows_v,
                out_hbm.at[pl.ds(base, chunk), pl.ds(i * emb_dim, emb_dim)])
            return carry

        lax.fori_loop(0, num_fields, field_body, 0)

    return sc_gather


def _mlp_body(cat_ref, num_ref, w1c_ref, w1n_ref, b1_ref, w2_ref, b2_ref,
              out_ref):
    h = jnp.dot(cat_ref[...], w1c_ref[...], preferred_element_type=jnp.float32)
    h = h + jnp.dot(num_ref[...], w1n_ref[...],
                    preferred_element_type=jnp.float32)
    h = jnp.maximum(h + b1_ref[...], 0.0)
    out_ref[...] = (
        jnp.dot(h, w2_ref[...], preferred_element_type=jnp.float32)
        + b2_ref[...]
    )


def _tc_mlp(cat_emb, x_num, w1c, w1n, b1, w2, b2, blk: int):
    b_rows = cat_emb.shape[0]
    grid = (b_rows // blk,)
    return pl.pallas_call(
        _mlp_body,
        grid=grid,
        in_specs=[
            pl.BlockSpec((blk, cat_emb.shape[1]), lambda i: (i, 0)),
            pl.BlockSpec((blk, x_num.shape[1]), lambda i: (i, 0)),
            pl.BlockSpec(w1c.shape, lambda i: (0, 0)),
            pl.BlockSpec(w1n.shape, lambda i: (0, 0)),
            pl.BlockSpec(b1.shape, lambda i: (0, 0)),
            pl.BlockSpec(w2.shape, lambda i: (0, 0)),
            pl.BlockSpec(b2.shape, lambda i: (0, 0)),
        ],
        out_specs=pl.BlockSpec((blk, 1), lambda i: (i, 0)),
        out_shape=jax.ShapeDtypeStruct((b_rows, 1), jnp.float32),
    )(cat_emb, x_num, w1c, w1n, b1, w2, b2)


def kernel(X_cat, X_num, tables, W1, b1, W2, b2):
    b_rows, num_fields = X_cat.shape
    vocab, emb_dim = tables.shape[1], tables.shape[2]

    tab_t = jnp.transpose(tables, (0, 2, 1))
    lin = _make_sc_linearize(num_fields, emb_dim, vocab)(tab_t)

    xcat_f = lax.bitcast_convert_type(X_cat.astype(jnp.int32), jnp.float32)
    cat_emb = _make_sc_gather(b_rows, num_fields, emb_dim, vocab)(
        lin.reshape(num_fields * vocab, emb_dim), xcat_f)

    w1c = W1[:, : num_fields * emb_dim].T
    w1n = W1[:, num_fields * emb_dim:].T
    out = _tc_mlp(cat_emb, X_num, w1c, w1n, b1[None, :], W2.T,
                  b2[None, :], blk=2048)
    return out[:, 0]


# speedup vs baseline: 1.2512x; 1.2512x over previous
import functools

import jax
import jax.numpy as jnp
from jax import lax
from jax.experimental import pallas as pl
from jax.experimental.pallas import tpu as pltpu
from jax.experimental.pallas import tpu_sc as plsc

NUM_CORES = 2
NUM_SUBCORES = 16
NW = NUM_CORES * NUM_SUBCORES
LANES = 16
NBUF = 8


def _make_sc_linearize(num_fields: int, emb_dim: int, vocab: int):
    full_tiles = vocab // 128
    tail = vocab - full_tiles * 128
    n_items = num_fields * full_tiles
    mesh = plsc.VectorSubcoreMesh(core_axis_name="c", subcore_axis_name="s")

    @functools.partial(
        pl.kernel,
        out_type=jax.ShapeDtypeStruct((num_fields * vocab * emb_dim,),
                                      jnp.float32),
        mesh=mesh,
        scratch_types=(
            [pltpu.VMEM((LANES, 128), jnp.float32) for _ in range(NBUF)]
            + [pltpu.VMEM((128 * LANES,), jnp.float32) for _ in range(NBUF)]
            + [pltpu.VMEM((LANES, 32), jnp.float32)]
            + [pltpu.SemaphoreType.DMA((NBUF,)),
               pltpu.SemaphoreType.DMA((NBUF,))]
        ),
        compiler_params=pltpu.CompilerParams(use_tc_tiling_on_sc=True,
                                             needs_layout_passes=False),
    )
    def sc_lin(tabt_hbm, out_hbm, *scr):
        sin = scr[:NBUF]
        sout = scr[NBUF:2 * NBUF]
        sin_t = scr[2 * NBUF]
        sem_i, sem_o = scr[2 * NBUF + 1], scr[2 * NBUF + 2]
        wid = lax.axis_index("s") * NUM_CORES + lax.axis_index("c")
        n_k = (n_items - wid + NW - 1) // NW
        lane = lax.iota(jnp.int32, LANES)
        lane16 = lane * emb_dim

        def item_fc(k):
            g = wid + k * NW
            return g // full_tiles, g % full_tiles

        def start_in(k, b):
            f, c = item_fc(k)
            pltpu.async_copy(tabt_hbm.at[f, :, pl.ds(c * 128, 128)],
                             sin[b], sem_i.at[b])

        def wait_in(b):
            pltpu.make_async_copy(tabt_hbm.at[0, :, pl.ds(0, 128)],
                                  sin[b], sem_i.at[b]).wait()

        def start_out(k, b):
            f, c = item_fc(k)
            off = (f * vocab + c * 128) * emb_dim
            pltpu.async_copy(sout[b], out_hbm.at[pl.ds(off, 128 * emb_dim)],
                             sem_o.at[b])

        def wait_out(b):
            pltpu.make_async_copy(sout[b],
                                  out_hbm.at[pl.ds(0, 128 * emb_dim)],
                                  sem_o.at[b]).wait()

        for b in range(NBUF):
            @pl.when(b < n_k)
            def _(b=b):
                start_in(b, b)

        def phase(k, b):
            @pl.when(k < n_k)
            def _():
                wait_in(b)

                @pl.when(k >= NBUF)
                def _():
                    wait_out(b)

                def t_body(t, carry):
                    t256 = t * (LANES * emb_dim)
                    for e in range(emb_dim):
                        vals = sin[b][e, pl.ds(t * LANES, LANES)]
                        plsc.store_scatter(sout[b], [lane16 + (t256 + e)],
                                           vals)
                    return carry

                lax.fori_loop(0, 128 // LANES, t_body, 0, unroll=2)
                start_out(k, b)

                @pl.when(k + NBUF < n_k)
                def _():
                    start_in(k + NBUF, b)

        def outer(k0, carry):
            for b in range(NBUF):
                phase(k0 * NBUF + b, b)
            return carry

        max_k = (n_items + NW - 1) // NW
        lax.fori_loop(0, (max_k + NBUF - 1) // NBUF, outer, 0)

        for b in range(NBUF):
            @pl.when(jnp.minimum(n_k, NBUF) > b)
            def _(b=b):
                wait_out(b)

        @pl.when(wid < num_fields)
        def _():
            f = wid
            pltpu.sync_copy(tabt_hbm.at[f, :, pl.ds(full_tiles * 128, tail)],
                            sin_t)

            def tt_body(t, carry):
                t256 = t * (LANES * emb_dim)
                for e in range(emb_dim):
                    vals = sin_t[e, pl.ds(t * LANES, LANES)]
                    plsc.store_scatter(sout[0], [lane16 + (t256 + e)], vals)
                return carry

            lax.fori_loop(0, tail // LANES, tt_body, 0)
            off = (f * vocab + full_tiles * 128) * emb_dim
            pltpu.sync_copy(sout[0].at[pl.ds(0, tail * emb_dim)],
                            out_hbm.at[pl.ds(off, tail * emb_dim)])

    return sc_lin


def _make_sc_gather(b_rows: int, num_fields: int, emb_dim: int, vocab: int):
    chunk = b_rows // NW
    mesh = plsc.VectorSubcoreMesh(core_axis_name="c", subcore_axis_name="s")

    @functools.partial(
        pl.kernel,
        out_type=jax.ShapeDtypeStruct((b_rows, num_fields * emb_dim),
                                      jnp.float32),
        mesh=mesh,
        scratch_types=[
            pltpu.VMEM((chunk, num_fields), jnp.float32),
            pltpu.VMEM((chunk,), jnp.int32),
            pltpu.VMEM((chunk, emb_dim), jnp.float32),
            pltpu.SemaphoreType.DMA,
        ],
        compiler_params=pltpu.CompilerParams(use_tc_tiling_on_sc=False,
                                             needs_layout_passes=False),
    )
    def sc_gather(tab_rows, xcat_hbm, out_hbm, xslab, idx_v, rows_v, sem):
        wid = lax.axis_index("s") * NUM_CORES + lax.axis_index("c")
        base = wid * chunk
        pltpu.sync_copy(xcat_hbm.at[pl.ds(base, chunk)], xslab)
        lane = lax.iota(jnp.int32, LANES)

        def field_body(i, carry):
            col = jnp.full((LANES,), i, jnp.int32)
            off = jnp.zeros((LANES,), jnp.int32) + i * vocab

            def extract_body(t, carry2):
                r16 = lane + t * LANES
                v = plsc.bitcast(plsc.load_gather(xslab, [r16, col]),
                                 jnp.int32)
                idx_v[pl.ds(t * LANES, LANES)] = v + off
                return carry2

            lax.fori_loop(0, chunk // LANES, extract_body, 0)
            pltpu.async_copy(tab_rows.at[idx_v], rows_v, sem).wait()
            pltpu.sync_copy(
                rows_v,
                out_hbm.at[pl.ds(base, chunk), pl.ds(i * emb_dim, emb_dim)])
            return carry

        lax.fori_loop(0, num_fields, field_body, 0)

    return sc_gather


def _mlp_body(cat_ref, num_ref, w1c_ref, w1n_ref, b1_ref, w2_ref, b2_ref,
              out_ref):
    h = jnp.dot(cat_ref[...], w1c_ref[...], preferred_element_type=jnp.float32)
    h = h + jnp.dot(num_ref[...], w1n_ref[...],
                    preferred_element_type=jnp.float32)
    h = jnp.maximum(h + b1_ref[...], 0.0)
    out_ref[...] = (
        jnp.dot(h, w2_ref[...], preferred_element_type=jnp.float32)
        + b2_ref[...]
    )


def _tc_mlp(cat_emb, x_num, w1c, w1n, b1, w2, b2, blk: int):
    b_rows = cat_emb.shape[0]
    grid = (b_rows // blk,)
    return pl.pallas_call(
        _mlp_body,
        grid=grid,
        in_specs=[
            pl.BlockSpec((blk, cat_emb.shape[1]), lambda i: (i, 0)),
            pl.BlockSpec((blk, x_num.shape[1]), lambda i: (i, 0)),
            pl.BlockSpec(w1c.shape, lambda i: (0, 0)),
            pl.BlockSpec(w1n.shape, lambda i: (0, 0)),
            pl.BlockSpec(b1.shape, lambda i: (0, 0)),
            pl.BlockSpec(w2.shape, lambda i: (0, 0)),
            pl.BlockSpec(b2.shape, lambda i: (0, 0)),
        ],
        out_specs=pl.BlockSpec((blk, 1), lambda i: (i, 0)),
        out_shape=jax.ShapeDtypeStruct((b_rows, 1), jnp.float32),
    )(cat_emb, x_num, w1c, w1n, b1, w2, b2)


def kernel(X_cat, X_num, tables, W1, b1, W2, b2):
    b_rows, num_fields = X_cat.shape
    vocab, emb_dim = tables.shape[1], tables.shape[2]

    tab_t = jnp.transpose(tables, (0, 2, 1))
    lin = _make_sc_linearize(num_fields, emb_dim, vocab)(tab_t)

    xcat_f = lax.bitcast_convert_type(X_cat.astype(jnp.int32), jnp.float32)
    cat_emb = _make_sc_gather(b_rows, num_fields, emb_dim, vocab)(
        lin.reshape(num_fields * vocab, emb_dim), xcat_f)

    w1c = W1[:, : num_fields * emb_dim].T
    w1n = W1[:, num_fields * emb_dim:].T
    out = _tc_mlp(cat_emb, X_num, w1c, w1n, b1[None, :], W2.T,
                  b2[None, :], blk=2048)
    return out[:, 0]
